# speedup vs baseline: 1.2265x; 1.2265x over previous
import functools

import jax
import jax.numpy as jnp
from jax import lax
from jax.experimental import pallas as pl
from jax.experimental.pallas import tpu as pltpu
from jax.experimental.pallas import tpu_sc as plsc

OUT_DIM = 128
BATCH = 1024
SEQ = 300

NUM_CORES = 2
NUM_SUBCORES = 16
NW = NUM_CORES * NUM_SUBCORES

B = BATCH * SEQ
B_PER_W = B // NW
CHUNK = 128
NCH = B_PER_W // CHUNK
NPAIR = NCH // 2


def _gather_rows(ids_flat, table):
    mesh = plsc.VectorSubcoreMesh(
        core_axis_name="c", subcore_axis_name="s",
        num_cores=NUM_CORES, num_subcores=NUM_SUBCORES)

    @functools.partial(
        pl.kernel,
        out_type=jax.ShapeDtypeStruct((B, OUT_DIM), jnp.float32),
        mesh=mesh,
        scratch_types=[
            pltpu.VMEM((B_PER_W,), jnp.int32),
            pltpu.VMEM((CHUNK, OUT_DIM), jnp.float32),
            pltpu.VMEM((CHUNK, OUT_DIM), jnp.float32),
            pltpu.SemaphoreType.DMA,
            pltpu.SemaphoreType.DMA,
        ],
    )
    def k(ids_hbm, table_hbm, out_hbm, idx_v, buf0, buf1, sem0, sem1):
        wid = lax.axis_index("s") * NUM_CORES + lax.axis_index("c")
        base = pl.multiple_of(wid * B_PER_W, CHUNK)
        pltpu.sync_copy(ids_hbm.at[pl.ds(base, B_PER_W)], idx_v)

        def start_gather(c, buf, sem):
            off = pl.multiple_of(c * CHUNK, CHUNK)
            pltpu.async_copy(table_hbm.at[idx_v.at[pl.ds(off, CHUNK)]], buf, sem)

        def wait_gather(buf, sem):
            pltpu.make_async_copy(table_hbm.at[pl.ds(0, CHUNK)], buf, sem).wait()

        def store(c, buf):
            off = pl.multiple_of(base + c * CHUNK, CHUNK)
            pltpu.sync_copy(buf, out_hbm.at[pl.ds(off, CHUNK)])

        start_gather(0, buf0, sem0)

        @pl.loop(0, NPAIR)
        def _(i):
            c0 = 2 * i
            start_gather(c0 + 1, buf1, sem1)
            wait_gather(buf0, sem0)
            store(c0, buf0)
            start_gather(c0 + 2, buf0, sem0)
            wait_gather(buf1, sem1)
            store(c0 + 1, buf1)

        wait_gather(buf0, sem0)
        store(NCH - 1, buf0)

    return k(ids_flat, table)


def kernel(input_ids, attention_mask, W):
    ids_flat = input_ids.reshape(-1)
    embeds = _gather_rows(ids_flat, W).reshape(BATCH, SEQ, OUT_DIM)
    return (embeds, embeds, attention_mask)
